# baseline (speedup 1.0000x reference)
import functools

import jax
import jax.numpy as jnp
from jax import lax
from jax.experimental import pallas as pl
from jax.experimental.pallas import tpu as pltpu
from jax.experimental.pallas import tpu_sc as plsc

L = 16
N = 1024
NV = N // L
CH = 3
ROW_W = N * CH
BITS = 6
BINS = 1 << BITS
PASSES = 6
CHUNK = N // L
PN = N + L


def _sc_body(pts_hbm, out_hbm,
             buf_in0, buf_out0, ka0, va0, kb0, vb0, hist0,
             buf_in1, buf_out1, ka1, va1, kb1, vb1, hist1,
             occ0, occ1, sem0, sem1, sem2, sem3,
             *, nc, rows_per_w):
    wid = lax.axis_index("s") * nc + lax.axis_index("c")
    lane = lax.broadcasted_iota(jnp.int32, (L,), 0)
    lane65 = lane * (CHUNK + 1)
    ones = jnp.ones((L,), jnp.int32)
    sign = jnp.full((L,), -(2 ** 31), jnp.int32)
    six = jnp.full((L,), 6, jnp.int32)

    def pad(pos):
        return pos + lax.shift_right_logical(pos, six)

    slot0 = (buf_in0, buf_out0, ka0, va0, kb0, vb0, hist0)
    slot1 = (buf_in1, buf_out1, ka1, va1, kb1, vb1, hist1)
    slots = (slot0, slot1)

    def do_pair(r, carry_row):
        row0 = wid * rows_per_w + 2 * r
        row1 = row0 + 1
        c0 = pltpu.async_copy(pts_hbm.at[row0], buf_in0, sem0)
        c1 = pltpu.async_copy(pts_hbm.at[row1], buf_in1, sem1)

        @pl.when(r > 0)
        def _():
            pltpu.make_async_copy(buf_out0, out_hbm.at[row0 - 2], sem2).wait()
            pltpu.make_async_copy(buf_out1, out_hbm.at[row1 - 2], sem3).wait()

        c0.wait()
        c1.wait()

        @plsc.parallel_loop(0, NV, unroll=8)
        def _build(v):
            i = lane + v * L
            pa = pad(i)
            i3 = i * CH
            for (b_in, _, ka, va, _, _, _) in slots:
                x = plsc.load_gather(b_in, [i3])
                k = plsc.bitcast(x, jnp.int32)
                ks = jnp.where(k < 0, ~k, k ^ sign)
                plsc.store_scatter(ka, [pa], ks)
                plsc.store_scatter(va, [pa], i)

        for p in range(PASSES):
            if p % 2 == 0:
                srcs = [(ka0, va0, kb0, vb0, hist0, occ0),
                        (ka1, va1, kb1, vb1, hist1, occ1)]
            else:
                srcs = [(kb0, vb0, ka0, va0, hist0, occ0),
                        (kb1, vb1, ka1, va1, hist1, occ1)]
            last = p == PASSES - 1
            shift = jnp.full((L,), p * BITS, jnp.int32)

            @plsc.parallel_loop(0, BINS, unroll=8)
            def _zero(v):
                z = jnp.zeros((L,), jnp.int32)
                hist0[pl.ds(v * L, L)] = z
                hist1[pl.ds(v * L, L)] = z

            G = 4

            def histo(g, c):
                j0 = g * G
                for (src_k, _, _, _, hi, ob) in srcs:
                    ds_ = []
                    addrs = []
                    for u in range(G):
                        k = plsc.load_gather(src_k, [lane65 + (j0 + u)])
                        d = lax.shift_right_logical(k, shift) & (BINS - 1)
                        ds_.append(d)
                        addrs.append(d * L + lane)
                    pre = [plsc.load_gather(hi, [a]) for a in addrs]
                    for u in range(G):
                        oc = pre[u]
                        for w in range(u):
                            oc = oc + jnp.where(ds_[u] == ds_[w], 1, 0)
                        plsc.store_scatter(ob, [lane65 + (j0 + u)], oc)
                    for u in range(G):
                        plsc.addupdate_scatter(hi, [addrs[u]], ones)
                return c

            lax.fori_loop(0, NV // G, histo, 0, unroll=4)

            @plsc.parallel_loop(0, BINS, unroll=4,
                                carry=(jnp.int32(0), jnp.int32(0)))
            def _scan(v, carry):
                ca, cb = carry
                h0 = hist0[pl.ds(v * L, L)]
                h1 = hist1[pl.ds(v * L, L)]
                inc0 = plsc.cumsum(h0)
                inc1 = plsc.cumsum(h1)
                hist0[pl.ds(v * L, L)] = inc0 - h0 + ca
                hist1[pl.ds(v * L, L)] = inc1 - h1 + cb
                return (ca + inc0[L - 1], cb + inc1[L - 1])

            if not last:

                @plsc.parallel_loop(0, NV, unroll=4)
                def _permute(j):
                    s = lane65 + j
                    for (src_k, src_v, dst_k, dst_v, hi, ob) in srcs:
                        k = plsc.load_gather(src_k, [s])
                        v = plsc.load_gather(src_v, [s])
                        oc = plsc.load_gather(ob, [s])
                        d = lax.shift_right_logical(k, shift) & (BINS - 1)
                        base = plsc.load_gather(hi, [d * L + lane])
                        pa = pad(base + oc)
                        plsc.store_scatter(dst_k, [pa], k)
                        plsc.store_scatter(dst_v, [pa], v)

            else:
                @plsc.parallel_loop(0, NV, unroll=4)
                def _permute_final(j):
                    s = lane65 + j
                    for (src_k, src_v, _, _, hi, ob), sl in zip(srcs, slots):
                        b_in, b_out = sl[0], sl[1]
                        k = plsc.load_gather(src_k, [s])
                        v = plsc.load_gather(src_v, [s])
                        oc = plsc.load_gather(ob, [s])
                        d = lax.shift_right_logical(k, shift) & (BINS - 1)
                        base = plsc.load_gather(hi, [d * L + lane])
                        off3 = (base + oc) * CH
                        v3 = v * CH
                        for ch in range(CH):
                            x = plsc.load_gather(b_in, [v3 + ch])
                            plsc.store_scatter(b_out, [off3 + ch], x)

        pltpu.async_copy(buf_out0, out_hbm.at[row0], sem2)
        pltpu.async_copy(buf_out1, out_hbm.at[row1], sem3)
        return carry_row

    lax.fori_loop(0, rows_per_w // 2, do_pair, 0)
    lastrow0 = wid * rows_per_w + rows_per_w - 2
    pltpu.make_async_copy(buf_out0, out_hbm.at[lastrow0], sem2).wait()
    pltpu.make_async_copy(buf_out1, out_hbm.at[lastrow0 + 1], sem3).wait()


def kernel(point_set, field_dims=3):
    b = 1
    for s in point_set.shape[:-2]:
        b *= s
    pts = point_set.reshape(b, ROW_W)
    info = plsc.get_sparse_core_info()
    nc = info.num_cores
    nw = nc * info.num_subcores
    rows_per_w = b // nw
    mesh = plsc.VectorSubcoreMesh(core_axis_name="c", subcore_axis_name="s")
    body = functools.partial(_sc_body, nc=nc, rows_per_w=rows_per_w)
    slot_types = [
        pltpu.VMEM((ROW_W,), jnp.float32),
        pltpu.VMEM((ROW_W,), jnp.float32),
        pltpu.VMEM((PN,), jnp.int32),
        pltpu.VMEM((PN,), jnp.int32),
        pltpu.VMEM((PN,), jnp.int32),
        pltpu.VMEM((PN,), jnp.int32),
        pltpu.VMEM((BINS * L,), jnp.int32),
    ]
    out = pl.kernel(
        body,
        out_type=jax.ShapeDtypeStruct((b, ROW_W), jnp.float32),
        mesh=mesh,
        compiler_params=pltpu.CompilerParams(needs_layout_passes=False),
        scratch_types=slot_types + slot_types + [
            pltpu.VMEM((PN,), jnp.int32),
            pltpu.VMEM((PN,), jnp.int32),
            pltpu.SemaphoreType.DMA,
            pltpu.SemaphoreType.DMA,
            pltpu.SemaphoreType.DMA,
            pltpu.SemaphoreType.DMA,
        ],
    )(pts)
    return out.reshape(point_set.shape)


# speedup vs baseline: 1.0588x; 1.0588x over previous
import functools

import jax
import jax.numpy as jnp
from jax import lax
from jax.experimental import pallas as pl
from jax.experimental.pallas import tpu as pltpu
from jax.experimental.pallas import tpu_sc as plsc

L = 16
N = 1024
NV = N // L
CH = 3
ROW_W = N * CH
BITS = 6
BINS = 1 << BITS
PASSES = 6
CHUNK = N // L
PN = N + L


def _sc_body(pts_hbm, out_hbm,
             buf_in0, buf_out0, ka0, va0, kb0, vb0, hist0,
             buf_in1, buf_out1, ka1, va1, kb1, vb1, hist1,
             occ0, occ1, sem0, sem1,
             *, nc, rows_per_w):
    wid = lax.axis_index("s") * nc + lax.axis_index("c")
    lane = lax.broadcasted_iota(jnp.int32, (L,), 0)
    lane65 = lane * (CHUNK + 1)
    ones = jnp.ones((L,), jnp.int32)
    sign = jnp.full((L,), -(2 ** 31), jnp.int32)
    six = jnp.full((L,), 6, jnp.int32)

    def pad(pos):
        return pos + lax.shift_right_logical(pos, six)

    slot0 = (buf_in0, buf_out0, ka0, va0, kb0, vb0, hist0)
    slot1 = (buf_in1, buf_out1, ka1, va1, kb1, vb1, hist1)
    slots = (slot0, slot1)

    def do_pair(r, carry_row):
        row0 = wid * rows_per_w + 2 * r
        row1 = row0 + 1
        c0 = pltpu.async_copy(pts_hbm.at[row0], buf_in0, sem0)
        c1 = pltpu.async_copy(pts_hbm.at[row1], buf_in1, sem1)
        c0.wait()
        c1.wait()

        @plsc.parallel_loop(0, NV, unroll=8)
        def _build(v):
            i = lane + v * L
            pa = pad(i)
            i3 = i * CH
            for (b_in, _, ka, va, _, _, _) in slots:
                x = plsc.load_gather(b_in, [i3])
                k = plsc.bitcast(x, jnp.int32)
                ks = jnp.where(k < 0, ~k, k ^ sign)
                plsc.store_scatter(ka, [pa], ks)
                plsc.store_scatter(va, [pa], i)

        for p in range(PASSES):
            if p % 2 == 0:
                srcs = [(ka0, va0, kb0, vb0, hist0, occ0),
                        (ka1, va1, kb1, vb1, hist1, occ1)]
            else:
                srcs = [(kb0, vb0, ka0, va0, hist0, occ0),
                        (kb1, vb1, ka1, va1, hist1, occ1)]
            shift = jnp.full((L,), p * BITS, jnp.int32)

            @plsc.parallel_loop(0, BINS, unroll=8)
            def _zero(v):
                z = jnp.zeros((L,), jnp.int32)
                hist0[pl.ds(v * L, L)] = z
                hist1[pl.ds(v * L, L)] = z

            G = 4

            def histo(g, c):
                j0 = g * G
                for (src_k, _, _, _, hi, ob) in srcs:
                    ds_ = []
                    addrs = []
                    for u in range(G):
                        k = plsc.load_gather(src_k, [lane65 + (j0 + u)])
                        d = lax.shift_right_logical(k, shift) & (BINS - 1)
                        ds_.append(d)
                        addrs.append(d * L + lane)
                    pre = [plsc.load_gather(hi, [a]) for a in addrs]
                    for u in range(G):
                        oc = pre[u]
                        for w in range(u):
                            oc = oc + jnp.where(ds_[u] == ds_[w], 1, 0)
                        plsc.store_scatter(ob, [lane65 + (j0 + u)], oc)
                    for u in range(G):
                        plsc.addupdate_scatter(hi, [addrs[u]], ones)
                return c

            lax.fori_loop(0, NV // G, histo, 0, unroll=4)

            @plsc.parallel_loop(0, BINS, unroll=4,
                                carry=(jnp.int32(0), jnp.int32(0)))
            def _scan(v, carry):
                ca, cb = carry
                h0 = hist0[pl.ds(v * L, L)]
                h1 = hist1[pl.ds(v * L, L)]
                inc0 = plsc.cumsum(h0)
                inc1 = plsc.cumsum(h1)
                hist0[pl.ds(v * L, L)] = inc0 - h0 + ca
                hist1[pl.ds(v * L, L)] = inc1 - h1 + cb
                return (ca + inc0[L - 1], cb + inc1[L - 1])

            @plsc.parallel_loop(0, NV, unroll=8)
            def _permute(j):
                s = lane65 + j
                for (src_k, src_v, dst_k, dst_v, hi, ob) in srcs:
                    k = plsc.load_gather(src_k, [s])
                    v = plsc.load_gather(src_v, [s])
                    oc = plsc.load_gather(ob, [s])
                    d = lax.shift_right_logical(k, shift) & (BINS - 1)
                    base = plsc.load_gather(hi, [d * L + lane])
                    pa = pad(base + oc)
                    plsc.store_scatter(dst_k, [pa], k)
                    plsc.store_scatter(dst_v, [pa], v)

        @plsc.parallel_loop(0, NV, unroll=4)
        def _gather_out(j):
            rr = lane + j * L
            par = pad(rr)
            r3 = rr * CH
            for (b_in, b_out, _, va, _, _, _) in slots:
                v = plsc.load_gather(va, [par])
                v3 = v * CH
                for ch in range(CH):
                    x = plsc.load_gather(b_in, [v3 + ch])
                    plsc.store_scatter(b_out, [r3 + ch], x)
        o0 = pltpu.async_copy(buf_out0, out_hbm.at[row0], sem0)
        o1 = pltpu.async_copy(buf_out1, out_hbm.at[row1], sem1)
        o0.wait()
        o1.wait()
        return carry_row

    lax.fori_loop(0, rows_per_w // 2, do_pair, 0)


def kernel(point_set, field_dims=3):
    b = 1
    for s in point_set.shape[:-2]:
        b *= s
    pts = point_set.reshape(b, ROW_W)
    info = plsc.get_sparse_core_info()
    nc = info.num_cores
    nw = nc * info.num_subcores
    rows_per_w = b // nw
    mesh = plsc.VectorSubcoreMesh(core_axis_name="c", subcore_axis_name="s")
    body = functools.partial(_sc_body, nc=nc, rows_per_w=rows_per_w)
    slot_types = [
        pltpu.VMEM((ROW_W,), jnp.float32),
        pltpu.VMEM((ROW_W,), jnp.float32),
        pltpu.VMEM((PN,), jnp.int32),
        pltpu.VMEM((PN,), jnp.int32),
        pltpu.VMEM((PN,), jnp.int32),
        pltpu.VMEM((PN,), jnp.int32),
        pltpu.VMEM((BINS * L,), jnp.int32),
    ]
    out = pl.kernel(
        body,
        out_type=jax.ShapeDtypeStruct((b, ROW_W), jnp.float32),
        mesh=mesh,
        compiler_params=pltpu.CompilerParams(needs_layout_passes=False),
        scratch_types=slot_types + slot_types + [
            pltpu.VMEM((PN,), jnp.int32),
            pltpu.VMEM((PN,), jnp.int32),
            pltpu.SemaphoreType.DMA,
            pltpu.SemaphoreType.DMA,
        ],
    )(pts)
    return out.reshape(point_set.shape)


# speedup vs baseline: 1.0794x; 1.0194x over previous
import functools

import jax
import jax.numpy as jnp
from jax import lax
from jax.experimental import pallas as pl
from jax.experimental.pallas import tpu as pltpu
from jax.experimental.pallas import tpu_sc as plsc

L = 16
N = 1024
NV = N // L
CH = 3
ROW_W = N * CH
BITS = 6
BINS = 1 << BITS
PASSES = 6
CHUNK = N // L
PN = N + L


def _sc_body(pts_hbm, out_hbm,
             buf_in0, buf_out0, ka0, va0, kb0, vb0, hist0,
             buf_in1, buf_out1, ka1, va1, kb1, vb1, hist1,
             occ0, occ1, sem0, sem1,
             *, nc, rows_per_w):
    wid = lax.axis_index("s") * nc + lax.axis_index("c")
    lane = lax.broadcasted_iota(jnp.int32, (L,), 0)
    lane65 = lane * (CHUNK + 1)
    ones = jnp.ones((L,), jnp.int32)
    sign = jnp.full((L,), -(2 ** 31), jnp.int32)
    six = jnp.full((L,), 6, jnp.int32)

    def pad(pos):
        return pos + lax.shift_right_logical(pos, six)

    slot0 = (buf_in0, buf_out0, ka0, va0, kb0, vb0, hist0)
    slot1 = (buf_in1, buf_out1, ka1, va1, kb1, vb1, hist1)
    slots = (slot0, slot1)

    def do_pair(r, carry_row):
        row0 = wid * rows_per_w + 2 * r
        row1 = row0 + 1
        c0 = pltpu.async_copy(pts_hbm.at[row0], buf_in0, sem0)
        c1 = pltpu.async_copy(pts_hbm.at[row1], buf_in1, sem1)
        c0.wait()
        c1.wait()

        @plsc.parallel_loop(0, NV, unroll=8)
        def _build(v):
            i = lane + v * L
            pa = pad(i)
            i3 = i * CH
            for (b_in, _, ka, va, _, _, _) in slots:
                x = plsc.load_gather(b_in, [i3])
                k = plsc.bitcast(x, jnp.int32)
                ks = jnp.where(k < 0, ~k, k ^ sign)
                plsc.store_scatter(ka, [pa], ks)
                plsc.store_scatter(va, [pa], i)

        for p in range(PASSES):
            if p % 2 == 0:
                srcs = [(ka0, va0, kb0, vb0, hist0, occ0),
                        (ka1, va1, kb1, vb1, hist1, occ1)]
            else:
                srcs = [(kb0, vb0, ka0, va0, hist0, occ0),
                        (kb1, vb1, ka1, va1, hist1, occ1)]
            shift = jnp.full((L,), p * BITS, jnp.int32)

            @plsc.parallel_loop(0, BINS, unroll=8)
            def _zero(v):
                z = jnp.zeros((L,), jnp.int32)
                hist0[pl.ds(v * L, L)] = z
                hist1[pl.ds(v * L, L)] = z

            G = 4

            def histo(g, c):
                j0 = g * G
                for (src_k, _, _, _, hi, ob) in srcs:
                    ds_ = []
                    addrs = []
                    for u in range(G):
                        k = plsc.load_gather(src_k, [lane65 + (j0 + u)])
                        d = lax.shift_right_logical(k, shift) & (BINS - 1)
                        ds_.append(d)
                        addrs.append(d * L + lane)
                    pre = [plsc.load_gather(hi, [a]) for a in addrs]
                    for u in range(G):
                        oc = pre[u]
                        for w in range(u):
                            oc = oc + jnp.where(ds_[u] == ds_[w], 1, 0)
                        plsc.store_scatter(ob, [lane65 + (j0 + u)], oc)
                    for u in range(G):
                        plsc.addupdate_scatter(hi, [addrs[u]], ones)
                return c

            lax.fori_loop(0, NV // G, histo, 0, unroll=4)

            @plsc.parallel_loop(0, BINS, unroll=4,
                                carry=(jnp.int32(0), jnp.int32(0)))
            def _scan(v, carry):
                ca, cb = carry
                h0 = hist0[pl.ds(v * L, L)]
                h1 = hist1[pl.ds(v * L, L)]
                inc0 = plsc.cumsum(h0)
                inc1 = plsc.cumsum(h1)
                hist0[pl.ds(v * L, L)] = inc0 - h0 + ca
                hist1[pl.ds(v * L, L)] = inc1 - h1 + cb
                return (ca + inc0[L - 1], cb + inc1[L - 1])

            @plsc.parallel_loop(0, NV, unroll=4)
            def _permute(j):
                s = lane65 + j
                for (src_k, src_v, dst_k, dst_v, hi, ob) in srcs:
                    k = plsc.load_gather(src_k, [s])
                    v = plsc.load_gather(src_v, [s])
                    oc = plsc.load_gather(ob, [s])
                    d = lax.shift_right_logical(k, shift) & (BINS - 1)
                    base = plsc.load_gather(hi, [d * L + lane])
                    pa = pad(base + oc)
                    plsc.store_scatter(dst_k, [pa], k)
                    plsc.store_scatter(dst_v, [pa], v)

        @plsc.parallel_loop(0, NV, unroll=4)
        def _gather_out(j):
            rr = lane + j * L
            par = pad(rr)
            r3 = rr * CH
            for (b_in, b_out, _, va, _, _, _) in slots:
                v = plsc.load_gather(va, [par])
                v3 = v * CH
                for ch in range(CH):
                    x = plsc.load_gather(b_in, [v3 + ch])
                    plsc.store_scatter(b_out, [r3 + ch], x)
        o0 = pltpu.async_copy(buf_out0, out_hbm.at[row0], sem0)
        o1 = pltpu.async_copy(buf_out1, out_hbm.at[row1], sem1)
        o0.wait()
        o1.wait()
        return carry_row

    lax.fori_loop(0, rows_per_w // 2, do_pair, 0)


def kernel(point_set, field_dims=3):
    b = 1
    for s in point_set.shape[:-2]:
        b *= s
    pts = point_set.reshape(b, ROW_W)
    info = plsc.get_sparse_core_info()
    nc = info.num_cores
    nw = nc * info.num_subcores
    rows_per_w = b // nw
    mesh = plsc.VectorSubcoreMesh(core_axis_name="c", subcore_axis_name="s")
    body = functools.partial(_sc_body, nc=nc, rows_per_w=rows_per_w)
    slot_types = [
        pltpu.VMEM((ROW_W,), jnp.float32),
        pltpu.VMEM((ROW_W,), jnp.float32),
        pltpu.VMEM((PN,), jnp.int32),
        pltpu.VMEM((PN,), jnp.int32),
        pltpu.VMEM((PN,), jnp.int32),
        pltpu.VMEM((PN,), jnp.int32),
        pltpu.VMEM((BINS * L,), jnp.int32),
    ]
    out = pl.kernel(
        body,
        out_type=jax.ShapeDtypeStruct((b, ROW_W), jnp.float32),
        mesh=mesh,
        compiler_params=pltpu.CompilerParams(needs_layout_passes=False),
        scratch_types=slot_types + slot_types + [
            pltpu.VMEM((PN,), jnp.int32),
            pltpu.VMEM((PN,), jnp.int32),
            pltpu.SemaphoreType.DMA,
            pltpu.SemaphoreType.DMA,
        ],
    )(pts)
    return out.reshape(point_set.shape)
